# baseline (speedup 1.0000x reference)
import jax
import jax.numpy as jnp
from jax.experimental import pallas as pl
from jax.experimental.pallas import tpu as pltpu

B, C, F = 4096, 62, 64
OUT = 64
R2, L = B * C // 2, 2 * F
BBB = 7936


def _body(x_ref, w2_ref, asrc_ref, adst_ref, bias_ref, bias2_ref, o_ref):
    hp = jnp.dot(x_ref[...], w2_ref[...],
                 preferred_element_type=jnp.float32)
    o_ref[...] = hp + bias2_ref[...]

    @pl.when(pl.program_id(0) == 0)
    def _attention():
        hp0 = hp[:32]
        hcat = jnp.concatenate([hp0[:, :F], hp0[:, F:]], axis=0)
        a_s = jnp.sum(hcat * asrc_ref[...], axis=1, keepdims=True)
        a_d = jnp.sum(hcat * adst_ref[...], axis=1, keepdims=True)
        e = a_s + a_d.reshape(1, 64)
        e = jnp.where(e >= 0, e, 0.2 * e)
        i_idx = jax.lax.broadcasted_iota(jnp.int32, (64, 64), 0)
        valid_i = (i_idx != 31) & (i_idx != 63)
        e = jnp.where(valid_i, e, -1e30)
        m = jnp.max(e, axis=0, keepdims=True)
        ex = jnp.where(valid_i, jnp.exp(e - m), 0.0)
        coef = ex / (jnp.sum(ex, axis=0, keepdims=True) + 1e-16)
        att = jax.lax.dot_general(
            coef, hcat, (((0,), (0,)), ((), ())),
            preferred_element_type=jnp.float32)
        j_idx = jax.lax.broadcasted_iota(jnp.int32, (64, 64), 0)
        valid_j = (j_idx != 31) & (j_idx != 63)
        res = jnp.where(valid_j, att, hcat) + bias_ref[...]
        o_ref[:32] = jnp.concatenate([res[:32], res[32:]], axis=1)


def kernel(x, W, att_src, att_dst, bias):
    x2 = x.reshape(R2, L)
    wt = W.T
    w2 = jnp.zeros((L, L), jnp.float32)
    w2 = w2.at[:F, :OUT].set(wt).at[F:, OUT:].set(wt)
    asrc = att_src.reshape(1, OUT)
    adst = att_dst.reshape(1, OUT)
    b1 = bias.reshape(1, OUT)
    b2 = jnp.concatenate([b1, b1], axis=1)
    out = pl.pallas_call(
        _body,
        grid=(R2 // BBB,),
        in_specs=[
            pl.BlockSpec((BBB, L), lambda i: (i, 0)),
            pl.BlockSpec((L, L), lambda i: (0, 0)),
            pl.BlockSpec((1, OUT), lambda i: (0, 0)),
            pl.BlockSpec((1, OUT), lambda i: (0, 0)),
            pl.BlockSpec((1, OUT), lambda i: (0, 0)),
            pl.BlockSpec((1, L), lambda i: (0, 0)),
        ],
        out_specs=pl.BlockSpec((BBB, L), lambda i: (i, 0)),
        out_shape=jax.ShapeDtypeStruct((R2, L), jnp.float32),
        compiler_params=pltpu.CompilerParams(
            dimension_semantics=("arbitrary",)),
    )(x2, w2, asrc, adst, b1, b2)
    return out.reshape(B, 1, C, F)


# speedup vs baseline: 1.4211x; 1.4211x over previous
import jax
import jax.numpy as jnp
from jax.experimental import pallas as pl
from jax.experimental.pallas import tpu as pltpu

B, C, F = 4096, 62, 64
OUT = 64
BB = 256


def _body(x_ref, wt_ref, asrc_ref, adst_ref, bias_ref, o_ref):
    bias = bias_ref[...]
    for b in range(BB):
        h = jnp.dot(x_ref[b, 0], wt_ref[...],
                    preferred_element_type=jnp.float32)
        if b == 0:
            @pl.when(pl.program_id(0) == 0)
            def _attention():
                a_s = jnp.sum(h * asrc_ref[...], axis=1, keepdims=True)
                a_d = jnp.sum(h * adst_ref[...], axis=1, keepdims=True)
                e = a_s + a_d.reshape(1, C)
                e = jnp.where(e >= 0, e, 0.2 * e)
                m = jnp.max(e, axis=0, keepdims=True)
                ex = jnp.exp(e - m)
                coef = ex / (jnp.sum(ex, axis=0, keepdims=True) + 1e-16)
                att = jax.lax.dot_general(
                    coef, h, (((0,), (0,)), ((), ())),
                    preferred_element_type=jnp.float32)
                o_ref[0, 0] = (att + bias).astype(jnp.bfloat16)

            @pl.when(pl.program_id(0) != 0)
            def _plain():
                o_ref[0, 0] = (h + bias).astype(jnp.bfloat16)
        else:
            o_ref[b, 0] = (h + bias).astype(jnp.bfloat16)


def kernel(x, W, att_src, att_dst, bias):
    x = x.astype(jnp.bfloat16)
    wt = W.T.astype(jnp.bfloat16)
    asrc = att_src.reshape(1, OUT)
    adst = att_dst.reshape(1, OUT)
    b2 = bias.reshape(1, OUT)
    return pl.pallas_call(
        _body,
        grid=(B // BB,),
        in_specs=[
            pl.BlockSpec((BB, 1, C, F), lambda i: (i, 0, 0, 0)),
            pl.BlockSpec((F, OUT), lambda i: (0, 0)),
            pl.BlockSpec((1, OUT), lambda i: (0, 0)),
            pl.BlockSpec((1, OUT), lambda i: (0, 0)),
            pl.BlockSpec((1, OUT), lambda i: (0, 0)),
        ],
        out_specs=pl.BlockSpec((BB, 1, C, OUT), lambda i: (i, 0, 0, 0)),
        out_shape=jax.ShapeDtypeStruct((B, 1, C, OUT), jnp.bfloat16),
        compiler_params=pltpu.CompilerParams(
            dimension_semantics=("parallel",)),
    )(x, wt, asrc, adst, b2).astype(jnp.float32)


# speedup vs baseline: 1.4276x; 1.0046x over previous
import jax
import jax.numpy as jnp
from jax.experimental import pallas as pl
from jax.experimental.pallas import tpu as pltpu

B, C, F = 4096, 62, 64
OUT = 64
BB = 512


def _body(x_ref, wt_ref, asrc_ref, adst_ref, bias_ref, o_ref):
    bias = bias_ref[...]
    for b in range(BB):
        h = jnp.dot(x_ref[b, 0], wt_ref[...],
                    preferred_element_type=jnp.float32)
        if b == 0:
            @pl.when(pl.program_id(0) == 0)
            def _attention():
                a_s = jnp.sum(h * asrc_ref[...], axis=1, keepdims=True)
                a_d = jnp.sum(h * adst_ref[...], axis=1, keepdims=True)
                e = a_s + a_d.reshape(1, C)
                e = jnp.where(e >= 0, e, 0.2 * e)
                m = jnp.max(e, axis=0, keepdims=True)
                ex = jnp.exp(e - m)
                coef = ex / (jnp.sum(ex, axis=0, keepdims=True) + 1e-16)
                att = jax.lax.dot_general(
                    coef, h, (((0,), (0,)), ((), ())),
                    preferred_element_type=jnp.float32)
                o_ref[0, 0] = (att + bias).astype(jnp.bfloat16)

            @pl.when(pl.program_id(0) != 0)
            def _plain():
                o_ref[0, 0] = (h + bias).astype(jnp.bfloat16)
        else:
            o_ref[b, 0] = (h + bias).astype(jnp.bfloat16)


def kernel(x, W, att_src, att_dst, bias):
    x = x.astype(jnp.bfloat16)
    wt = W.T.astype(jnp.bfloat16)
    asrc = att_src.reshape(1, OUT)
    adst = att_dst.reshape(1, OUT)
    b2 = bias.reshape(1, OUT)
    return pl.pallas_call(
        _body,
        grid=(B // BB,),
        in_specs=[
            pl.BlockSpec((BB, 1, C, F), lambda i: (i, 0, 0, 0)),
            pl.BlockSpec((F, OUT), lambda i: (0, 0)),
            pl.BlockSpec((1, OUT), lambda i: (0, 0)),
            pl.BlockSpec((1, OUT), lambda i: (0, 0)),
            pl.BlockSpec((1, OUT), lambda i: (0, 0)),
        ],
        out_specs=pl.BlockSpec((BB, 1, C, OUT), lambda i: (i, 0, 0, 0)),
        out_shape=jax.ShapeDtypeStruct((B, 1, C, OUT), jnp.bfloat16),
        compiler_params=pltpu.CompilerParams(
            dimension_semantics=("parallel",)),
    )(x, wt, asrc, adst, b2).astype(jnp.float32)
